# baseline (speedup 1.0000x reference)
import functools

import jax
import jax.numpy as jnp
from jax import lax
from jax.experimental import pallas as pl
from jax.experimental.pallas import tpu as pltpu
from jax.experimental.pallas import tpu_sc as plsc

N = 50000
CH = 128
NBATCH = 4
H = 256
W = 256
EPS = 1e-5

BLK = 512
NP = 50176
NSTEP = NP // BLK
PH = H + 2
PW = W + 2
PHW = PH * PW
GW = NBATCH * PHW
GA = GW + 8
TRASH = GW

NW = 32
NGRP = NP // 128
CAP = 1024
HDR = 128
RW = HDR + CAP + 128
GDMA = 128

OFFS = tuple(dy * PW + dx for dy in (-1, 0, 1) for dx in (-1, 0, 1))


def _wid():
    return lax.axis_index("s") * 2 + lax.axis_index("c")


def _sc_mesh():
    return plsc.VectorSubcoreMesh(
        core_axis_name="c", subcore_axis_name="s",
        num_cores=2, num_subcores=16)


@functools.cache
def _get_sc_scatter():
    @functools.partial(
        pl.kernel,
        out_type=(jax.ShapeDtypeStruct((GA,), jnp.int32),
                  jax.ShapeDtypeStruct((NP,), jnp.int32)),
        mesh=_sc_mesh(),
        scratch_types=[
            pltpu.VMEM((3, 128), jnp.int32),
            pltpu.VMEM((128,), jnp.int32),
            pltpu.VMEM((128,), jnp.int32),
            pltpu.SemaphoreType.DMA,
        ],
    )
    def sc_scatter(ind_hbm, grid_hbm, pos_hbm, indv, fbuf, vbuf, sem):
        wid = _wid()

        @pl.loop(0, (NGRP + NW - 1) // NW)
        def _t(t):
            g = wid + t * NW

            @pl.when(g < NGRP)
            def _():
                r0 = g * 128
                pltpu.sync_copy(ind_hbm.at[:, pl.ds(r0, 128)], indv)
                for u in range(8):
                    s = pl.ds(u * 16, 16)
                    b = indv[0, s]
                    y = indv[1, s]
                    x = indv[2, s]
                    fl = b * PHW + (y + 1) * PW + (x + 1)
                    rid = r0 + u * 16 + lax.iota(jnp.int32, 16)
                    fbuf[s] = jnp.where(rid < N, fl, TRASH)
                    vbuf[s] = rid
                pltpu.sync_copy(fbuf, pos_hbm.at[pl.ds(r0, 128)])
                pltpu.async_copy(vbuf, grid_hbm.at[fbuf], sem).wait()

    return sc_scatter


@functools.cache
def _get_sc_nbr():
    @functools.partial(
        pl.kernel,
        out_type=jax.ShapeDtypeStruct((NGRP * RW,), jnp.int32),
        mesh=_sc_mesh(),
        scratch_types=[
            pltpu.VMEM((3, 128), jnp.int32),
            pltpu.VMEM((128,), jnp.int32),
            pltpu.VMEM((8, 128), jnp.int32),
            pltpu.VMEM((8, 128), jnp.int32),
            pltpu.VMEM((8, 128), jnp.int32),
            pltpu.VMEM((8, 128), jnp.int32),
            pltpu.VMEM((RW,), jnp.int32),
            pltpu.VMEM((16,), jnp.int32),
        ] + [pltpu.VMEM((128,), jnp.int32) for _ in range(16)] + [
            pltpu.SemaphoreType.DMA,
        ],
    )
    def sc_nbr(ind_hbm, grid_hbm, pos_hbm, cmp_hbm,
               indv, fbuf, nf8, v8, c8, p8, zbuf, hdrb, *rest):
        valbs = rest[0:8]
        posbs = rest[8:16]
        sem = rest[16]
        wid = _wid()
        iot = lax.iota(jnp.int32, 16)
        zpk = jnp.full((16,), N * 128, jnp.int32)

        @pl.loop(0, RW // 16)
        def _z(i):
            zbuf[pl.ds(i * 16, 16)] = zpk

        ks = [k for k in range(9) if k != 4]

        @pl.loop(0, (NGRP + NW - 1) // NW)
        def _t(t):
            g = wid + t * NW

            @pl.when(g < NGRP)
            def _():
                r0 = g * 128
                rowb = g * RW
                pltpu.sync_copy(zbuf, cmp_hbm.at[pl.ds(rowb, RW)])
                pltpu.sync_copy(ind_hbm.at[:, pl.ds(r0, 128)], indv)
                for u in range(8):
                    s = pl.ds(u * 16, 16)
                    fbuf[s] = (indv[0, s] * PHW + (indv[1, s] + 1) * PW
                               + (indv[2, s] + 1))
                for i, k in enumerate(ks):
                    for u in range(8):
                        s = pl.ds(u * 16, 16)
                        nf8[i, s] = fbuf[s] + OFFS[k]
                ds1 = [pltpu.async_copy(grid_hbm.at[nf8.at[i]], v8.at[i],
                                        sem) for i in range(8)]
                for d in ds1:
                    d.wait()
                for i in range(8):
                    for u in range(8):
                        s = pl.ds(u * 16, 16)
                        c8[i, s] = jnp.clip(v8[i, s], 0, N - 1)
                ds2 = [pltpu.async_copy(pos_hbm.at[c8.at[i]], p8.at[i],
                                        sem) for i in range(8)]
                for d in ds2:
                    d.wait()

                cnt = jnp.int32(0)
                for i, k in enumerate(ks):
                    for u in range(8):
                        s = pl.ds(u * 16, 16)
                        v = v8[i, s]
                        rid = r0 + u * 16 + iot
                        valid = ((v >= 0) & (v < N) & (p8[i, s] == nf8[i, s])
                                 & (rid < N))
                        valbs[i][s] = jnp.where(
                            valid, (v + k * NP) * 128 + (u * 16 + iot),
                            N * 128)
                ds3 = [pltpu.async_copy(
                    valbs[i], cmp_hbm.at[pl.ds(rowb + HDR + i * 128, 128)],
                    sem) for i in range(8)]
                for d in ds3:
                    d.wait()
                cnt = jnp.int32(CAP)
                cntp = (cnt + (GDMA - 1)) & ~jnp.int32(GDMA - 1)
                hdrb[pl.ds(0, 16)] = jnp.zeros((16,), jnp.int32) + cntp
                pltpu.sync_copy(hdrb, cmp_hbm.at[pl.ds(rowb, 16)])

    return sc_nbr


@functools.cache
def _get_sc_conv():
    @functools.partial(
        pl.kernel,
        out_type=jax.ShapeDtypeStruct((NP, CH), jnp.float32),
        mesh=_sc_mesh(),
        scratch_types=[
            pltpu.VMEM((1, RW), jnp.int32),
            pltpu.VMEM((GDMA,), jnp.int32),
            pltpu.VMEM((GDMA, CH), jnp.float32),
            pltpu.VMEM((128, CH), jnp.float32),
            pltpu.SemaphoreType.DMA,
            pltpu.SemaphoreType.DMA,
        ],
    )
    def sc_conv(cmp_hbm, tbl_hbm, out_hbm, crow, idxb, gbuf, obuf, sem, osem):
        wid = _wid()

        @pl.loop(0, (NGRP + NW - 1) // NW)
        def _t(t):
            g = wid + t * NW

            @pl.when(g < NGRP)
            def _():
                r0 = g * 128
                dc = pltpu.async_copy(
                    tbl_hbm.at[pl.ds(4 * NP + r0, 128), :], obuf, osem)
                pltpu.sync_copy(cmp_hbm.at[pl.ds(g, 1), :], crow)
                cnt = jnp.clip(crow[0, pl.ds(0, 16)][0], 0, CAP)
                dc.wait()

                @pl.loop(0, cnt // GDMA)
                def _b(bi):
                    base = HDR + bi * GDMA
                    for q in range(GDMA // 16):
                        p16 = crow[0, pl.ds(base + q * 16, 16)]
                        idxb[pl.ds(q * 16, 16)] = jnp.clip(
                            lax.shift_right_logical(p16, 7), 0, 9 * NP - 1)
                    pltpu.async_copy(
                        tbl_hbm.at[idxb], gbuf, sem).wait()

                    @pl.loop(0, GDMA // 16)
                    def _q(q):
                        d16 = crow[0, pl.ds(base + q * 16, 16)] & 127
                        for l in range(16):
                            m = q * 16 + l
                            dstm = d16[l]
                            for c in range(CH // 16):
                                s = pl.ds(c * 16, 16)
                                obuf[dstm, s] = obuf[dstm, s] + gbuf[m, s]

                pltpu.sync_copy(obuf, out_hbm.at[pl.ds(r0, 128), :])

    return sc_conv


def _sc_scatter(ind_t):
    return _get_sc_scatter()(ind_t)


def _sc_nbr(ind_t, grid, pos):
    return _get_sc_nbr()(ind_t, grid, pos)


def _sc_conv(cmp, tbl):
    return _get_sc_conv()(cmp.reshape(NGRP, RW), tbl)


def _bnstats_body(x_ref, g_ref, b_ref, o_ref, acc_ref):
    m = pl.program_id(0)

    @pl.when(m == 0)
    def _():
        acc_ref[...] = jnp.zeros_like(acc_ref)

    x = x_ref[...]
    acc_ref[0:1, :] += jnp.sum(x, axis=0)[None, :]
    acc_ref[1:2, :] += jnp.sum(x * x, axis=0)[None, :]

    @pl.when(m == NSTEP - 1)
    def _():
        mean = acc_ref[0:1, :] / N
        var = acc_ref[1:2, :] / N - mean * mean
        scale = g_ref[0:1, :] * lax.rsqrt(var + EPS)
        o_ref[0:1, :] = scale
        o_ref[1:2, :] = b_ref[0:1, :] - mean * scale


def _tc_bnstats(x, gamma8, beta8):
    return pl.pallas_call(
        _bnstats_body,
        grid=(NSTEP,),
        in_specs=[
            pl.BlockSpec((BLK, CH), lambda m: (m, 0)),
            pl.BlockSpec((8, CH), lambda m: (0, 0)),
            pl.BlockSpec((8, CH), lambda m: (0, 0)),
        ],
        out_specs=pl.BlockSpec((8, CH), lambda m: (0, 0)),
        out_shape=jax.ShapeDtypeStruct((8, CH), jnp.float32),
        scratch_shapes=[pltpu.VMEM((8, CH), jnp.float32)],
    )(x, gamma8, beta8)


def _bnmm_body(x_ref, ss_ref, w_ref, o_ref):
    m = pl.program_id(0)
    x = x_ref[...]
    scale = ss_ref[0:1, :]
    shift = ss_ref[1:2, :]
    rid = lax.broadcasted_iota(jnp.int32, (BLK, 1), 0) + m * BLK
    xb = jnp.where(rid < N, jnp.maximum(x * scale + shift, 0.0), 0.0)
    for k in range(9):
        o_ref[k] = jnp.dot(xb, w_ref[k], preferred_element_type=jnp.float32)


def _tc_bnmm(x, ss, w):
    return pl.pallas_call(
        _bnmm_body,
        grid=(NSTEP,),
        in_specs=[
            pl.BlockSpec((BLK, CH), lambda m: (m, 0)),
            pl.BlockSpec((8, CH), lambda m: (0, 0)),
            pl.BlockSpec((9, CH, CH), lambda m: (0, 0, 0)),
        ],
        out_specs=pl.BlockSpec((9, BLK, CH), lambda m: (0, m, 0)),
        out_shape=jax.ShapeDtypeStruct((9, NP, CH), jnp.float32),
    )(x, ss, w)


def _se_body(x_ref, b_ref, w1_ref, b1_ref, w2_ref, b2_ref, o_ref,
             accp_ref, accc_ref):
    m = pl.program_id(0)

    @pl.when(m == 0)
    def _():
        accp_ref[...] = jnp.zeros_like(accp_ref)
        accc_ref[...] = jnp.zeros_like(accc_ref)

    x = x_ref[...]
    b = b_ref[...]
    for j in range(NBATCH):
        mj = b == j
        accp_ref[j:j + 1, :] += jnp.sum(jnp.where(mj, x, 0.0), axis=0)[None, :]
        accc_ref[j:j + 1, :] += jnp.sum(mj.astype(jnp.float32))

    @pl.when(m == NSTEP - 1)
    def _():
        avg = accp_ref[...] / jnp.maximum(accc_ref[...], 1.0)
        h = jnp.dot(avg, w1_ref[...], preferred_element_type=jnp.float32)
        h = jnp.maximum(h + b1_ref[...], 0.0)
        se = jnp.dot(h, w2_ref[...], preferred_element_type=jnp.float32)
        o_ref[...] = jax.nn.sigmoid(se + b2_ref[...])


def _tc_se(x, bcol, w1t, b1, w2t, b2):
    red = w1t.shape[1]
    return pl.pallas_call(
        _se_body,
        grid=(NSTEP,),
        in_specs=[
            pl.BlockSpec((BLK, CH), lambda m: (m, 0)),
            pl.BlockSpec((BLK, 1), lambda m: (m, 0)),
            pl.BlockSpec((CH, red), lambda m: (0, 0)),
            pl.BlockSpec((8, red), lambda m: (0, 0)),
            pl.BlockSpec((red, CH), lambda m: (0, 0)),
            pl.BlockSpec((8, CH), lambda m: (0, 0)),
        ],
        out_specs=pl.BlockSpec((8, CH), lambda m: (0, 0)),
        out_shape=jax.ShapeDtypeStruct((8, CH), jnp.float32),
        scratch_shapes=[
            pltpu.VMEM((8, CH), jnp.float32),
            pltpu.VMEM((8, CH), jnp.float32),
        ],
    )(x, bcol, w1t, b1, w2t, b2)


def _final_body(x_ref, f_ref, b_ref, se_ref, o_ref):
    x = x_ref[...]
    b = b_ref[...]
    sel = jnp.zeros((BLK, CH), jnp.float32)
    for j in range(NBATCH):
        sel = sel + jnp.where(b == j, se_ref[j:j + 1, :], 0.0)
    o_ref[...] = x * sel + f_ref[...]


def _tc_final(x, feats_pad, bcol, se):
    return pl.pallas_call(
        _final_body,
        grid=(NSTEP,),
        in_specs=[
            pl.BlockSpec((BLK, CH), lambda m: (m, 0)),
            pl.BlockSpec((BLK, CH), lambda m: (m, 0)),
            pl.BlockSpec((BLK, 1), lambda m: (m, 0)),
            pl.BlockSpec((8, CH), lambda m: (0, 0)),
        ],
        out_specs=pl.BlockSpec((BLK, CH), lambda m: (m, 0)),
        out_shape=jax.ShapeDtypeStruct((NP, CH), jnp.float32),
    )(x, feats_pad, bcol, se)


def kernel(feats, indices, gamma1, beta1, W1, gamma2, beta2, W2,
           fc1_w, fc1_b, fc2_w, fc2_b):
    feats_pad = jnp.pad(feats, ((0, NP - N), (0, 0)))
    ind_t = jnp.pad(indices, ((0, NP - N), (0, 0))).T
    bcol = jnp.pad(indices[:, 0:1], ((0, NP - N), (0, 0)),
                   constant_values=-1)

    gamma18 = jnp.broadcast_to(gamma1[None, :], (8, CH))
    beta18 = jnp.broadcast_to(beta1[None, :], (8, CH))
    gamma28 = jnp.broadcast_to(gamma2[None, :], (8, CH))
    beta28 = jnp.broadcast_to(beta2[None, :], (8, CH))
    red = fc1_w.shape[0]
    fc1_wt = fc1_w.T
    fc1_b8 = jnp.broadcast_to(fc1_b[None, :], (8, red))
    fc2_wt = fc2_w.T
    fc2_b8 = jnp.broadcast_to(fc2_b[None, :], (8, CH))

    grid, pos = _sc_scatter(ind_t)
    cmp = _sc_nbr(ind_t, grid, pos)

    ss1 = _tc_bnstats(feats_pad, gamma18, beta18)
    tmp1 = _tc_bnmm(feats_pad, ss1, W1)
    out1 = _sc_conv(cmp, tmp1.reshape(9 * NP, CH))

    ss2 = _tc_bnstats(out1, gamma28, beta28)
    tmp2 = _tc_bnmm(out1, ss2, W2)
    out2 = _sc_conv(cmp, tmp2.reshape(9 * NP, CH))

    se = _tc_se(out2, bcol, fc1_wt, fc1_b8, fc2_wt, fc2_b8)
    outp = _tc_final(out2, feats_pad, bcol, se)
    return outp[:N]


# speedup vs baseline: 5.2858x; 5.2858x over previous
import functools

import jax
import jax.numpy as jnp
from jax import lax
from jax.experimental import pallas as pl
from jax.experimental.pallas import tpu as pltpu
from jax.experimental.pallas import tpu_sc as plsc

N = 50000
CH = 128
NBATCH = 4
H = 256
W = 256
EPS = 1e-5

BLK = 512
NP = 50176
NSTEP = NP // BLK
PH = H + 2
PW = W + 2
PHW = PH * PW
GW = NBATCH * PHW
GA = GW + 8
TRASH = GW

NW = 32
NGRP = NP // 128
CCH = 64
NCHUNK = NP // CCH

OFFS = tuple(dy * PW + dx for dy in (-1, 0, 1) for dx in (-1, 0, 1))


def _wid():
    return lax.axis_index("s") * 2 + lax.axis_index("c")


def _sc_mesh():
    return plsc.VectorSubcoreMesh(
        core_axis_name="c", subcore_axis_name="s",
        num_cores=2, num_subcores=16)


@functools.cache
def _get_sc_scatter():
    @functools.partial(
        pl.kernel,
        out_type=(jax.ShapeDtypeStruct((GA,), jnp.int32),
                  jax.ShapeDtypeStruct((NP,), jnp.int32)),
        mesh=_sc_mesh(),
        scratch_types=[
            pltpu.VMEM((3, 128), jnp.int32),
            pltpu.VMEM((128,), jnp.int32),
            pltpu.VMEM((128,), jnp.int32),
            pltpu.SemaphoreType.DMA,
        ],
    )
    def sc_scatter(ind_hbm, grid_hbm, pos_hbm, indv, fbuf, vbuf, sem):
        wid = _wid()

        @pl.loop(0, (NGRP + NW - 1) // NW)
        def _t(t):
            g = wid + t * NW

            @pl.when(g < NGRP)
            def _():
                r0 = g * 128
                pltpu.sync_copy(ind_hbm.at[:, pl.ds(r0, 128)], indv)
                for u in range(8):
                    s = pl.ds(u * 16, 16)
                    b = indv[0, s]
                    y = indv[1, s]
                    x = indv[2, s]
                    fl = b * PHW + (y + 1) * PW + (x + 1)
                    rid = r0 + u * 16 + lax.iota(jnp.int32, 16)
                    fbuf[s] = jnp.where(rid < N, fl, TRASH)
                    vbuf[s] = rid
                pltpu.sync_copy(fbuf, pos_hbm.at[pl.ds(r0, 128)])
                pltpu.async_copy(vbuf, grid_hbm.at[fbuf], sem).wait()

    return sc_scatter


@functools.cache
def _get_sc_nbr():
    @functools.partial(
        pl.kernel,
        out_type=jax.ShapeDtypeStruct((9, NP), jnp.int32),
        mesh=_sc_mesh(),
        scratch_types=[
            pltpu.VMEM((3, 128), jnp.int32),
            pltpu.VMEM((128,), jnp.int32),
            pltpu.VMEM((8, 128), jnp.int32),
            pltpu.VMEM((8, 128), jnp.int32),
            pltpu.VMEM((8, 128), jnp.int32),
            pltpu.VMEM((8, 128), jnp.int32),
            pltpu.VMEM((1, 128), jnp.int32),
            pltpu.SemaphoreType.DMA,
        ],
    )
    def sc_nbr(ind_hbm, grid_hbm, pos_hbm, nbr_hbm,
               indv, fbuf, nf8, v8, c8, p8, obuf, sem):
        wid = _wid()
        iot = lax.iota(jnp.int32, 16)
        ks = [k for k in range(9) if k != 4]

        @pl.loop(0, (NGRP + NW - 1) // NW)
        def _t(t):
            g = wid + t * NW

            @pl.when(g < NGRP)
            def _():
                r0 = g * 128
                pltpu.sync_copy(ind_hbm.at[:, pl.ds(r0, 128)], indv)
                for u in range(8):
                    s = pl.ds(u * 16, 16)
                    fbuf[s] = (indv[0, s] * PHW + (indv[1, s] + 1) * PW
                               + (indv[2, s] + 1))
                for i, k in enumerate(ks):
                    for u in range(8):
                        s = pl.ds(u * 16, 16)
                        nf8[i, s] = fbuf[s] + OFFS[k]
                ds1 = [pltpu.async_copy(grid_hbm.at[nf8.at[i]], v8.at[i],
                                        sem) for i in range(8)]
                for d in ds1:
                    d.wait()
                for i in range(8):
                    for u in range(8):
                        s = pl.ds(u * 16, 16)
                        c8[i, s] = jnp.clip(v8[i, s], 0, N - 1)
                ds2 = [pltpu.async_copy(pos_hbm.at[c8.at[i]], p8.at[i],
                                        sem) for i in range(8)]
                for d in ds2:
                    d.wait()
                for u in range(8):
                    s = pl.ds(u * 16, 16)
                    obuf[0, s] = 4 * NP + r0 + u * 16 + iot
                pltpu.sync_copy(obuf, nbr_hbm.at[pl.ds(4, 1), pl.ds(r0, 128)])
                for i, k in enumerate(ks):
                    for u in range(8):
                        s = pl.ds(u * 16, 16)
                        v = v8[i, s]
                        rid = r0 + u * 16 + iot
                        valid = ((v >= 0) & (v < N) & (p8[i, s] == nf8[i, s])
                                 & (rid < N))
                        obuf[0, s] = jnp.where(valid, v, N) + k * NP
                    pltpu.sync_copy(
                        obuf, nbr_hbm.at[pl.ds(k, 1), pl.ds(r0, 128)])

    return sc_nbr


@functools.cache
def _get_sc_conv():
    @functools.partial(
        pl.kernel,
        out_type=jax.ShapeDtypeStruct((NP, CH), jnp.float32),
        mesh=_sc_mesh(),
        scratch_types=[
            pltpu.VMEM((9, 128), jnp.int32),
            pltpu.VMEM((9, CCH, CH), jnp.float32),
            pltpu.VMEM((CCH, CH), jnp.float32),
            pltpu.SemaphoreType.DMA,
        ],
    )
    def sc_conv(nbr_hbm, tbl_hbm, out_hbm, idxv, bufs, obuf, sem):
        wid = _wid()

        @pl.loop(0, (NGRP + NW - 1) // NW)
        def _t(t):
            g = wid + t * NW

            @pl.when(g < NGRP)
            def _():
                r0 = g * 128
                pltpu.sync_copy(nbr_hbm.at[:, pl.ds(r0, 128)], idxv)
                for h in range(2):
                    descs = [
                        pltpu.async_copy(
                            tbl_hbm.at[idxv.at[k, pl.ds(h * CCH, CCH)]],
                            bufs.at[k], sem)
                        for k in range(9)
                    ]
                    for d in descs:
                        d.wait()

                    @pl.loop(0, CCH)
                    def _r(r):
                        for u in range(CH // 16):
                            s = pl.ds(u * 16, 16)
                            acc = bufs[0, r, s]
                            for k in range(1, 9):
                                acc = acc + bufs[k, r, s]
                            obuf[r, s] = acc

                    pltpu.sync_copy(
                        obuf, out_hbm.at[pl.ds(r0 + h * CCH, CCH), :])

    return sc_conv


def _sc_scatter(ind_t):
    return _get_sc_scatter()(ind_t)


def _sc_nbr(ind_t, grid, pos):
    return _get_sc_nbr()(ind_t, grid, pos)


def _sc_conv(nbr, tbl):
    return _get_sc_conv()(nbr, tbl)


def _bnstats_body(x_ref, g_ref, b_ref, o_ref, acc_ref):
    m = pl.program_id(0)

    @pl.when(m == 0)
    def _():
        acc_ref[...] = jnp.zeros_like(acc_ref)

    x = x_ref[...]
    acc_ref[0:1, :] += jnp.sum(x, axis=0)[None, :]
    acc_ref[1:2, :] += jnp.sum(x * x, axis=0)[None, :]

    @pl.when(m == NSTEP - 1)
    def _():
        mean = acc_ref[0:1, :] / N
        var = acc_ref[1:2, :] / N - mean * mean
        scale = g_ref[0:1, :] * lax.rsqrt(var + EPS)
        o_ref[0:1, :] = scale
        o_ref[1:2, :] = b_ref[0:1, :] - mean * scale


def _tc_bnstats(x, gamma8, beta8):
    return pl.pallas_call(
        _bnstats_body,
        grid=(NSTEP,),
        in_specs=[
            pl.BlockSpec((BLK, CH), lambda m: (m, 0)),
            pl.BlockSpec((8, CH), lambda m: (0, 0)),
            pl.BlockSpec((8, CH), lambda m: (0, 0)),
        ],
        out_specs=pl.BlockSpec((8, CH), lambda m: (0, 0)),
        out_shape=jax.ShapeDtypeStruct((8, CH), jnp.float32),
        scratch_shapes=[pltpu.VMEM((8, CH), jnp.float32)],
    )(x, gamma8, beta8)


def _bnmm_body(x_ref, ss_ref, w_ref, o_ref):
    m = pl.program_id(0)
    x = x_ref[...]
    scale = ss_ref[0:1, :]
    shift = ss_ref[1:2, :]
    rid = lax.broadcasted_iota(jnp.int32, (BLK, 1), 0) + m * BLK
    xb = jnp.where(rid < N, jnp.maximum(x * scale + shift, 0.0), 0.0)
    for k in range(9):
        o_ref[k] = jnp.dot(xb, w_ref[k], preferred_element_type=jnp.float32)


def _tc_bnmm(x, ss, w):
    return pl.pallas_call(
        _bnmm_body,
        grid=(NSTEP,),
        in_specs=[
            pl.BlockSpec((BLK, CH), lambda m: (m, 0)),
            pl.BlockSpec((8, CH), lambda m: (0, 0)),
            pl.BlockSpec((9, CH, CH), lambda m: (0, 0, 0)),
        ],
        out_specs=pl.BlockSpec((9, BLK, CH), lambda m: (0, m, 0)),
        out_shape=jax.ShapeDtypeStruct((9, NP, CH), jnp.float32),
    )(x, ss, w)


def _se_body(x_ref, b_ref, w1_ref, b1_ref, w2_ref, b2_ref, o_ref,
             accp_ref, accc_ref):
    m = pl.program_id(0)

    @pl.when(m == 0)
    def _():
        accp_ref[...] = jnp.zeros_like(accp_ref)
        accc_ref[...] = jnp.zeros_like(accc_ref)

    x = x_ref[...]
    b = b_ref[...]
    for j in range(NBATCH):
        mj = b == j
        accp_ref[j:j + 1, :] += jnp.sum(jnp.where(mj, x, 0.0), axis=0)[None, :]
        accc_ref[j:j + 1, :] += jnp.sum(mj.astype(jnp.float32))

    @pl.when(m == NSTEP - 1)
    def _():
        avg = accp_ref[...] / jnp.maximum(accc_ref[...], 1.0)
        h = jnp.dot(avg, w1_ref[...], preferred_element_type=jnp.float32)
        h = jnp.maximum(h + b1_ref[...], 0.0)
        se = jnp.dot(h, w2_ref[...], preferred_element_type=jnp.float32)
        o_ref[...] = jax.nn.sigmoid(se + b2_ref[...])


def _tc_se(x, bcol, w1t, b1, w2t, b2):
    red = w1t.shape[1]
    return pl.pallas_call(
        _se_body,
        grid=(NSTEP,),
        in_specs=[
            pl.BlockSpec((BLK, CH), lambda m: (m, 0)),
            pl.BlockSpec((BLK, 1), lambda m: (m, 0)),
            pl.BlockSpec((CH, red), lambda m: (0, 0)),
            pl.BlockSpec((8, red), lambda m: (0, 0)),
            pl.BlockSpec((red, CH), lambda m: (0, 0)),
            pl.BlockSpec((8, CH), lambda m: (0, 0)),
        ],
        out_specs=pl.BlockSpec((8, CH), lambda m: (0, 0)),
        out_shape=jax.ShapeDtypeStruct((8, CH), jnp.float32),
        scratch_shapes=[
            pltpu.VMEM((8, CH), jnp.float32),
            pltpu.VMEM((8, CH), jnp.float32),
        ],
    )(x, bcol, w1t, b1, w2t, b2)


def _final_body(x_ref, f_ref, b_ref, se_ref, o_ref):
    x = x_ref[...]
    b = b_ref[...]
    sel = jnp.zeros((BLK, CH), jnp.float32)
    for j in range(NBATCH):
        sel = sel + jnp.where(b == j, se_ref[j:j + 1, :], 0.0)
    o_ref[...] = x * sel + f_ref[...]


def _tc_final(x, feats_pad, bcol, se):
    return pl.pallas_call(
        _final_body,
        grid=(NSTEP,),
        in_specs=[
            pl.BlockSpec((BLK, CH), lambda m: (m, 0)),
            pl.BlockSpec((BLK, CH), lambda m: (m, 0)),
            pl.BlockSpec((BLK, 1), lambda m: (m, 0)),
            pl.BlockSpec((8, CH), lambda m: (0, 0)),
        ],
        out_specs=pl.BlockSpec((BLK, CH), lambda m: (m, 0)),
        out_shape=jax.ShapeDtypeStruct((NP, CH), jnp.float32),
    )(x, feats_pad, bcol, se)


def kernel(feats, indices, gamma1, beta1, W1, gamma2, beta2, W2,
           fc1_w, fc1_b, fc2_w, fc2_b):
    feats_pad = jnp.pad(feats, ((0, NP - N), (0, 0)))
    ind_t = jnp.pad(indices, ((0, NP - N), (0, 0))).T
    bcol = jnp.pad(indices[:, 0:1], ((0, NP - N), (0, 0)),
                   constant_values=-1)

    gamma18 = jnp.broadcast_to(gamma1[None, :], (8, CH))
    beta18 = jnp.broadcast_to(beta1[None, :], (8, CH))
    gamma28 = jnp.broadcast_to(gamma2[None, :], (8, CH))
    beta28 = jnp.broadcast_to(beta2[None, :], (8, CH))
    red = fc1_w.shape[0]
    fc1_wt = fc1_w.T
    fc1_b8 = jnp.broadcast_to(fc1_b[None, :], (8, red))
    fc2_wt = fc2_w.T
    fc2_b8 = jnp.broadcast_to(fc2_b[None, :], (8, CH))

    grid, pos = _sc_scatter(ind_t)
    nbr = _sc_nbr(ind_t, grid, pos)

    ss1 = _tc_bnstats(feats_pad, gamma18, beta18)
    tmp1 = _tc_bnmm(feats_pad, ss1, W1)
    out1 = _sc_conv(nbr, tmp1.reshape(9 * NP, CH))

    ss2 = _tc_bnstats(out1, gamma28, beta28)
    tmp2 = _tc_bnmm(out1, ss2, W2)
    out2 = _sc_conv(nbr, tmp2.reshape(9 * NP, CH))

    se = _tc_se(out2, bcol, fc1_wt, fc1_b8, fc2_wt, fc2_b8)
    outp = _tc_final(out2, feats_pad, bcol, se)
    return outp[:N]


# speedup vs baseline: 5.3334x; 1.0090x over previous
import functools

import jax
import jax.numpy as jnp
from jax import lax
from jax.experimental import pallas as pl
from jax.experimental.pallas import tpu as pltpu
from jax.experimental.pallas import tpu_sc as plsc

N = 50000
CH = 128
NBATCH = 4
H = 256
W = 256
EPS = 1e-5

BLK = 512
NP = 50176
NSTEP = NP // BLK
PH = H + 2
PW = W + 2
PHW = PH * PW
GW = NBATCH * PHW
GA = GW + 8
TRASH = GW

NW = 32
NGRP = NP // 128
CCH = 64
NCHUNK = NP // CCH

OFFS = tuple(dy * PW + dx for dy in (-1, 0, 1) for dx in (-1, 0, 1))


def _wid():
    return lax.axis_index("s") * 2 + lax.axis_index("c")


def _sc_mesh():
    return plsc.VectorSubcoreMesh(
        core_axis_name="c", subcore_axis_name="s",
        num_cores=2, num_subcores=16)


@functools.cache
def _get_sc_scatter():
    @functools.partial(
        pl.kernel,
        out_type=(jax.ShapeDtypeStruct((GA,), jnp.int32),
                  jax.ShapeDtypeStruct((NP,), jnp.int32)),
        mesh=_sc_mesh(),
        scratch_types=[
            pltpu.VMEM((3, 128), jnp.int32),
            pltpu.VMEM((128,), jnp.int32),
            pltpu.VMEM((128,), jnp.int32),
            pltpu.SemaphoreType.DMA,
        ],
    )
    def sc_scatter(ind_hbm, grid_hbm, pos_hbm, indv, fbuf, vbuf, sem):
        wid = _wid()

        @pl.loop(0, (NGRP + NW - 1) // NW)
        def _t(t):
            g = wid + t * NW

            @pl.when(g < NGRP)
            def _():
                r0 = g * 128
                pltpu.sync_copy(ind_hbm.at[:, pl.ds(r0, 128)], indv)
                for u in range(8):
                    s = pl.ds(u * 16, 16)
                    b = indv[0, s]
                    y = indv[1, s]
                    x = indv[2, s]
                    fl = b * PHW + (y + 1) * PW + (x + 1)
                    rid = r0 + u * 16 + lax.iota(jnp.int32, 16)
                    fbuf[s] = jnp.where(rid < N, fl, TRASH)
                    vbuf[s] = rid
                pltpu.sync_copy(fbuf, pos_hbm.at[pl.ds(r0, 128)])
                pltpu.async_copy(vbuf, grid_hbm.at[fbuf], sem).wait()

    return sc_scatter


@functools.cache
def _get_sc_nbr():
    @functools.partial(
        pl.kernel,
        out_type=jax.ShapeDtypeStruct((9, NP), jnp.int32),
        mesh=_sc_mesh(),
        scratch_types=[
            pltpu.VMEM((3, 128), jnp.int32),
            pltpu.VMEM((128,), jnp.int32),
            pltpu.VMEM((8, 128), jnp.int32),
            pltpu.VMEM((8, 128), jnp.int32),
            pltpu.VMEM((8, 128), jnp.int32),
            pltpu.VMEM((8, 128), jnp.int32),
            pltpu.VMEM((1, 128), jnp.int32),
            pltpu.SemaphoreType.DMA,
        ],
    )
    def sc_nbr(ind_hbm, grid_hbm, pos_hbm, nbr_hbm,
               indv, fbuf, nf8, v8, c8, p8, obuf, sem):
        wid = _wid()
        iot = lax.iota(jnp.int32, 16)
        ks = [k for k in range(9) if k != 4]

        @pl.loop(0, (NGRP + NW - 1) // NW)
        def _t(t):
            g = wid + t * NW

            @pl.when(g < NGRP)
            def _():
                r0 = g * 128
                pltpu.sync_copy(ind_hbm.at[:, pl.ds(r0, 128)], indv)
                for u in range(8):
                    s = pl.ds(u * 16, 16)
                    fbuf[s] = (indv[0, s] * PHW + (indv[1, s] + 1) * PW
                               + (indv[2, s] + 1))
                for i, k in enumerate(ks):
                    for u in range(8):
                        s = pl.ds(u * 16, 16)
                        nf8[i, s] = fbuf[s] + OFFS[k]
                ds1 = [pltpu.async_copy(grid_hbm.at[nf8.at[i]], v8.at[i],
                                        sem) for i in range(8)]
                for d in ds1:
                    d.wait()
                for i in range(8):
                    for u in range(8):
                        s = pl.ds(u * 16, 16)
                        c8[i, s] = jnp.clip(v8[i, s], 0, N - 1)
                ds2 = [pltpu.async_copy(pos_hbm.at[c8.at[i]], p8.at[i],
                                        sem) for i in range(8)]
                for d in ds2:
                    d.wait()
                for u in range(8):
                    s = pl.ds(u * 16, 16)
                    obuf[0, s] = 4 * NP + r0 + u * 16 + iot
                pltpu.sync_copy(obuf, nbr_hbm.at[pl.ds(4, 1), pl.ds(r0, 128)])
                for i, k in enumerate(ks):
                    for u in range(8):
                        s = pl.ds(u * 16, 16)
                        v = v8[i, s]
                        rid = r0 + u * 16 + iot
                        valid = ((v >= 0) & (v < N) & (p8[i, s] == nf8[i, s])
                                 & (rid < N))
                        obuf[0, s] = jnp.where(valid, v, N) + k * NP
                    pltpu.sync_copy(
                        obuf, nbr_hbm.at[pl.ds(k, 1), pl.ds(r0, 128)])

    return sc_nbr


@functools.cache
def _get_sc_conv():
    @functools.partial(
        pl.kernel,
        out_type=jax.ShapeDtypeStruct((NP, CH), jnp.float32),
        mesh=_sc_mesh(),
        scratch_types=[
            pltpu.VMEM((9, 128), jnp.int32),
            pltpu.VMEM((8, CCH, CH), jnp.float32),
            pltpu.VMEM((128, CH), jnp.float32),
            pltpu.SemaphoreType.DMA,
            pltpu.SemaphoreType.DMA,
        ],
    )
    def sc_conv(nbr_hbm, tbl_hbm, out_hbm, idxv, bufs, obuf, sem, osem):
        wid = _wid()
        ks = [k for k in range(9) if k != 4]

        @pl.loop(0, (NGRP + NW - 1) // NW)
        def _t(t):
            g = wid + t * NW

            @pl.when(g < NGRP)
            def _():
                r0 = g * 128
                dc = pltpu.async_copy(
                    tbl_hbm.at[pl.ds(4 * NP + r0, 128), :], obuf, osem)
                pltpu.sync_copy(nbr_hbm.at[:, pl.ds(r0, 128)], idxv)
                dc.wait()
                for h in range(2):
                    descs = [
                        pltpu.async_copy(
                            tbl_hbm.at[idxv.at[k, pl.ds(h * CCH, CCH)]],
                            bufs.at[i], sem)
                        for i, k in enumerate(ks)
                    ]
                    for d in descs:
                        d.wait()

                    @pl.loop(0, CCH)
                    def _r(r):
                        for u in range(CH // 16):
                            s = pl.ds(u * 16, 16)
                            acc = obuf[h * CCH + r, s]
                            for i in range(8):
                                acc = acc + bufs[i, r, s]
                            obuf[h * CCH + r, s] = acc

                pltpu.sync_copy(obuf, out_hbm.at[pl.ds(r0, 128), :])

    return sc_conv


def _sc_scatter(ind_t):
    return _get_sc_scatter()(ind_t)


def _sc_nbr(ind_t, grid, pos):
    return _get_sc_nbr()(ind_t, grid, pos)


def _sc_conv(nbr, tbl):
    return _get_sc_conv()(nbr, tbl)


def _bnstats_body(x_ref, g_ref, b_ref, o_ref, acc_ref):
    m = pl.program_id(0)

    @pl.when(m == 0)
    def _():
        acc_ref[...] = jnp.zeros_like(acc_ref)

    x = x_ref[...]
    acc_ref[0:1, :] += jnp.sum(x, axis=0)[None, :]
    acc_ref[1:2, :] += jnp.sum(x * x, axis=0)[None, :]

    @pl.when(m == NSTEP - 1)
    def _():
        mean = acc_ref[0:1, :] / N
        var = acc_ref[1:2, :] / N - mean * mean
        scale = g_ref[0:1, :] * lax.rsqrt(var + EPS)
        o_ref[0:1, :] = scale
        o_ref[1:2, :] = b_ref[0:1, :] - mean * scale


def _tc_bnstats(x, gamma8, beta8):
    return pl.pallas_call(
        _bnstats_body,
        grid=(NSTEP,),
        in_specs=[
            pl.BlockSpec((BLK, CH), lambda m: (m, 0)),
            pl.BlockSpec((8, CH), lambda m: (0, 0)),
            pl.BlockSpec((8, CH), lambda m: (0, 0)),
        ],
        out_specs=pl.BlockSpec((8, CH), lambda m: (0, 0)),
        out_shape=jax.ShapeDtypeStruct((8, CH), jnp.float32),
        scratch_shapes=[pltpu.VMEM((8, CH), jnp.float32)],
    )(x, gamma8, beta8)


def _bnmm_body(x_ref, ss_ref, w_ref, o_ref):
    m = pl.program_id(0)
    x = x_ref[...]
    scale = ss_ref[0:1, :]
    shift = ss_ref[1:2, :]
    rid = lax.broadcasted_iota(jnp.int32, (BLK, 1), 0) + m * BLK
    xb = jnp.where(rid < N, jnp.maximum(x * scale + shift, 0.0), 0.0)
    for k in range(9):
        o_ref[k] = jnp.dot(xb, w_ref[k], preferred_element_type=jnp.float32)


def _tc_bnmm(x, ss, w):
    return pl.pallas_call(
        _bnmm_body,
        grid=(NSTEP,),
        in_specs=[
            pl.BlockSpec((BLK, CH), lambda m: (m, 0)),
            pl.BlockSpec((8, CH), lambda m: (0, 0)),
            pl.BlockSpec((9, CH, CH), lambda m: (0, 0, 0)),
        ],
        out_specs=pl.BlockSpec((9, BLK, CH), lambda m: (0, m, 0)),
        out_shape=jax.ShapeDtypeStruct((9, NP, CH), jnp.float32),
    )(x, ss, w)


def _se_body(x_ref, b_ref, w1_ref, b1_ref, w2_ref, b2_ref, o_ref,
             accp_ref, accc_ref):
    m = pl.program_id(0)

    @pl.when(m == 0)
    def _():
        accp_ref[...] = jnp.zeros_like(accp_ref)
        accc_ref[...] = jnp.zeros_like(accc_ref)

    x = x_ref[...]
    b = b_ref[...]
    for j in range(NBATCH):
        mj = b == j
        accp_ref[j:j + 1, :] += jnp.sum(jnp.where(mj, x, 0.0), axis=0)[None, :]
        accc_ref[j:j + 1, :] += jnp.sum(mj.astype(jnp.float32))

    @pl.when(m == NSTEP - 1)
    def _():
        avg = accp_ref[...] / jnp.maximum(accc_ref[...], 1.0)
        h = jnp.dot(avg, w1_ref[...], preferred_element_type=jnp.float32)
        h = jnp.maximum(h + b1_ref[...], 0.0)
        se = jnp.dot(h, w2_ref[...], preferred_element_type=jnp.float32)
        o_ref[...] = jax.nn.sigmoid(se + b2_ref[...])


def _tc_se(x, bcol, w1t, b1, w2t, b2):
    red = w1t.shape[1]
    return pl.pallas_call(
        _se_body,
        grid=(NSTEP,),
        in_specs=[
            pl.BlockSpec((BLK, CH), lambda m: (m, 0)),
            pl.BlockSpec((BLK, 1), lambda m: (m, 0)),
            pl.BlockSpec((CH, red), lambda m: (0, 0)),
            pl.BlockSpec((8, red), lambda m: (0, 0)),
            pl.BlockSpec((red, CH), lambda m: (0, 0)),
            pl.BlockSpec((8, CH), lambda m: (0, 0)),
        ],
        out_specs=pl.BlockSpec((8, CH), lambda m: (0, 0)),
        out_shape=jax.ShapeDtypeStruct((8, CH), jnp.float32),
        scratch_shapes=[
            pltpu.VMEM((8, CH), jnp.float32),
            pltpu.VMEM((8, CH), jnp.float32),
        ],
    )(x, bcol, w1t, b1, w2t, b2)


def _final_body(x_ref, f_ref, b_ref, se_ref, o_ref):
    x = x_ref[...]
    b = b_ref[...]
    sel = jnp.zeros((BLK, CH), jnp.float32)
    for j in range(NBATCH):
        sel = sel + jnp.where(b == j, se_ref[j:j + 1, :], 0.0)
    o_ref[...] = x * sel + f_ref[...]


def _tc_final(x, feats_pad, bcol, se):
    return pl.pallas_call(
        _final_body,
        grid=(NSTEP,),
        in_specs=[
            pl.BlockSpec((BLK, CH), lambda m: (m, 0)),
            pl.BlockSpec((BLK, CH), lambda m: (m, 0)),
            pl.BlockSpec((BLK, 1), lambda m: (m, 0)),
            pl.BlockSpec((8, CH), lambda m: (0, 0)),
        ],
        out_specs=pl.BlockSpec((BLK, CH), lambda m: (m, 0)),
        out_shape=jax.ShapeDtypeStruct((NP, CH), jnp.float32),
    )(x, feats_pad, bcol, se)


def kernel(feats, indices, gamma1, beta1, W1, gamma2, beta2, W2,
           fc1_w, fc1_b, fc2_w, fc2_b):
    feats_pad = jnp.pad(feats, ((0, NP - N), (0, 0)))
    ind_t = jnp.pad(indices, ((0, NP - N), (0, 0))).T
    bcol = jnp.pad(indices[:, 0:1], ((0, NP - N), (0, 0)),
                   constant_values=-1)

    gamma18 = jnp.broadcast_to(gamma1[None, :], (8, CH))
    beta18 = jnp.broadcast_to(beta1[None, :], (8, CH))
    gamma28 = jnp.broadcast_to(gamma2[None, :], (8, CH))
    beta28 = jnp.broadcast_to(beta2[None, :], (8, CH))
    red = fc1_w.shape[0]
    fc1_wt = fc1_w.T
    fc1_b8 = jnp.broadcast_to(fc1_b[None, :], (8, red))
    fc2_wt = fc2_w.T
    fc2_b8 = jnp.broadcast_to(fc2_b[None, :], (8, CH))

    grid, pos = _sc_scatter(ind_t)
    nbr = _sc_nbr(ind_t, grid, pos)

    ss1 = _tc_bnstats(feats_pad, gamma18, beta18)
    tmp1 = _tc_bnmm(feats_pad, ss1, W1)
    out1 = _sc_conv(nbr, tmp1.reshape(9 * NP, CH))

    ss2 = _tc_bnstats(out1, gamma28, beta28)
    tmp2 = _tc_bnmm(out1, ss2, W2)
    out2 = _sc_conv(nbr, tmp2.reshape(9 * NP, CH))

    se = _tc_se(out2, bcol, fc1_wt, fc1_b8, fc2_wt, fc2_b8)
    outp = _tc_final(out2, feats_pad, bcol, se)
    return outp[:N]
